# baseline (speedup 1.0000x reference)
import jax
import jax.numpy as jnp
from jax import lax
from jax.experimental import pallas as pl
from jax.experimental.pallas import tpu as pltpu
from jax.experimental.pallas import tpu_sc as plsc

N_NODES = 10000
N_EDGES = 320000
D = 128
NC = 2
NS = 16
NW = NC * NS
NPAD = 10240
EPW = N_EDGES // NW
EPWP = 10240
CHK = 200
RZ = 200
RPS = NPAD // NS


def _mesh():
    return plsc.VectorSubcoreMesh(core_axis_name="c", subcore_axis_name="s")


def _deg_body(dst_hbm, out_hbm, dbuf, ones, zv, shared):
    c = lax.axis_index("c")
    s = lax.axis_index("s")
    w = c * NS + s

    def fill(i, _):
        ones[pl.ds(i * 16, 16)] = jnp.full((16,), 1.0, jnp.float32)
        zv[pl.ds(i * 16, 16)] = jnp.zeros((16,), jnp.float32)
        return 0

    lax.fori_loop(0, NPAD // 16, fill, 0)

    @pl.when(s == 0)
    def _():
        pltpu.sync_copy(zv, shared)

    plsc.subcore_barrier()
    pltpu.sync_copy(dst_hbm.at[pl.ds(w * EPW, EPW)], dbuf)
    pltpu.sync_copy(ones.at[pl.ds(0, EPW)], shared.at[dbuf], add=True)
    plsc.subcore_barrier()

    @pl.when(s == 0)
    def _():
        pltpu.sync_copy(shared, out_hbm.at[c])


def _k_deg(dst):
    f = pl.kernel(
        _deg_body,
        out_type=jax.ShapeDtypeStruct((NC, NPAD), jnp.float32),
        mesh=_mesh(),
        scratch_types=[
            pltpu.VMEM((EPW,), jnp.int32),
            pltpu.VMEM((NPAD,), jnp.float32),
            pltpu.VMEM((NPAD,), jnp.float32),
            pltpu.VMEM_SHARED((NPAD,), jnp.float32),
        ],
    )
    return f(dst)


HCHK = 128
NCH = EPWP // HCHK
LOOPN = NCH // 2
TAIL = NCH % 2


def _hop_body(u_hbm, src_hbm, dst_hbm, out_hbm, rows0, rows1, sb0, sb1,
              db0, db1, shared, gsem0, gsem1, ssem0, ssem1, isem0, isem1):
    c = lax.axis_index("c")
    s = lax.axis_index("s")
    w = c * NS + s
    ebase = w * EPWP
    zf = jnp.zeros((16,), jnp.float32)
    rbufs = (rows0, rows1)
    sbufs = (sb0, sb1)
    dbufs = (db0, db1)
    gsems = (gsem0, gsem1)
    ssems = (ssem0, ssem1)
    isems = (isem0, isem1)

    def zr(r, _):
        for f in range(D // 16):
            rows0[r, pl.ds(f * 16, 16)] = zf
        return 0

    lax.fori_loop(0, HCHK, zr, 0)

    def zs(i, _):
        pltpu.sync_copy(rows0, shared.at[pl.ds(s * RPS + i * HCHK, HCHK)])
        return 0

    lax.fori_loop(0, RPS // HCHK, zs, 0)
    plsc.subcore_barrier()

    def _idx_start(k, b):
        pltpu.async_copy(src_hbm.at[pl.ds(ebase + k * HCHK, HCHK)],
                         sbufs[b], isems[b])
        pltpu.async_copy(dst_hbm.at[pl.ds(ebase + k * HCHK, HCHK)],
                         dbufs[b], isems[b])

    def _gather_start(b):
        return pltpu.async_copy(u_hbm.at[sbufs[b]], rbufs[b], gsems[b])

    def _gather_wait(b):
        pltpu.make_async_copy(u_hbm.at[sbufs[b]], rbufs[b], gsems[b]).wait()

    def _scatter_start(b):
        return pltpu.async_copy(rbufs[b], shared.at[dbufs[b]], ssems[b],
                                add=True)

    def _scatter_wait(b):
        pltpu.make_async_copy(rbufs[b], shared.at[dbufs[b]], ssems[b]).wait()

    def _idx_wait(b):
        pltpu.make_async_copy(src_hbm.at[pl.ds(0, HCHK)], sbufs[b],
                              isems[b]).wait()
        pltpu.make_async_copy(dst_hbm.at[pl.ds(0, HCHK)], dbufs[b],
                              isems[b]).wait()

    _idx_start(0, 0)
    _idx_wait(0)
    _gather_start(0)
    _idx_start(1, 1)
    _idx_wait(1)
    _gather_start(1)

    def body(t, _):
        _gather_wait(0)
        _scatter_start(0)
        _gather_wait(1)
        _scatter_start(1)

        @pl.when(t < LOOPN - 1)
        def _():
            _scatter_wait(0)
            _idx_start(2 * t + 2, 0)
            _idx_wait(0)
            _gather_start(0)
            _scatter_wait(1)
            _idx_start(2 * t + 3, 1)
            _idx_wait(1)
            _gather_start(1)

        @pl.when(t >= LOOPN - 1)
        def _():
            _scatter_wait(0)
            _scatter_wait(1)

        return 0

    lax.fori_loop(0, LOOPN, body, 0)
    if TAIL:
        _idx_start(NCH - 1, 0)
        _idx_wait(0)
        _gather_start(0)
        _gather_wait(0)
        _scatter_start(0)
        _scatter_wait(0)
    plsc.subcore_barrier()
    pltpu.sync_copy(shared.at[pl.ds(s * RPS, RPS)],
                    out_hbm.at[c, pl.ds(s * RPS, RPS)])


def _k_hop(u, src, dst):
    f = pl.kernel(
        _hop_body,
        out_type=jax.ShapeDtypeStruct((NC, NPAD, D), jnp.float32),
        mesh=_mesh(),
        scratch_types=[
            pltpu.VMEM((HCHK, D), jnp.float32),
            pltpu.VMEM((HCHK, D), jnp.float32),
            pltpu.VMEM((HCHK,), jnp.int32),
            pltpu.VMEM((HCHK,), jnp.int32),
            pltpu.VMEM((HCHK,), jnp.int32),
            pltpu.VMEM((HCHK,), jnp.int32),
            pltpu.VMEM_SHARED((NPAD, D), jnp.float32),
            pltpu.SemaphoreType.DMA,
            pltpu.SemaphoreType.DMA,
            pltpu.SemaphoreType.DMA,
            pltpu.SemaphoreType.DMA,
            pltpu.SemaphoreType.DMA,
            pltpu.SemaphoreType.DMA,
        ],
    )
    return f(u, src, dst)


def _prep_kernel(x_ref, dp_ref, u_ref, dis_ref, inv_ref):
    deg = dp_ref[0] + dp_ref[1] + 1.0
    dis = lax.rsqrt(deg)
    dis_ref[...] = dis
    inv_ref[...] = 1.0 / deg
    u_ref[...] = x_ref[...] * dis[:, None]


def _k_prep(xp, degp):
    return pl.pallas_call(
        _prep_kernel,
        out_shape=[
            jax.ShapeDtypeStruct((NPAD, D), jnp.float32),
            jax.ShapeDtypeStruct((NPAD,), jnp.float32),
            jax.ShapeDtypeStruct((NPAD,), jnp.float32),
        ],
    )(xp, degp)


def _mid_kernel(p_ref, u_ref, inv_ref, w_ref):
    w_ref[...] = (p_ref[0] + p_ref[1] + u_ref[...]) * inv_ref[...][:, None]


def _k_mid(p, u, invdeg):
    return pl.pallas_call(
        _mid_kernel,
        out_shape=jax.ShapeDtypeStruct((NPAD, D), jnp.float32),
    )(p, u, invdeg)


def _fin_kernel(q_ref, w_ref, dis_ref, wt_ref, b_ref, o_ref):
    h2 = (q_ref[0] + q_ref[1] + w_ref[...]) * dis_ref[...][:, None]
    o_ref[...] = (
        jnp.dot(h2, wt_ref[...], preferred_element_type=jnp.float32)
        + b_ref[...]
    )


def _k_fin(q, w1, dis, W, b):
    return pl.pallas_call(
        _fin_kernel,
        out_shape=jax.ShapeDtypeStruct((NPAD, D), jnp.float32),
    )(q, w1, dis, W.T, b.reshape(1, D))


def kernel(x, edge_index, W, b):
    ei = edge_index.astype(jnp.int32)
    xp = jnp.pad(x, ((0, NPAD - N_NODES), (0, 0)))
    pad = NPAD - 1
    src = jnp.pad(ei[0].reshape(NW, EPW), ((0, 0), (0, EPWP - EPW)),
                  constant_values=pad).reshape(-1)
    dst = jnp.pad(ei[1].reshape(NW, EPW), ((0, 0), (0, EPWP - EPW)),
                  constant_values=pad).reshape(-1)
    degp = _k_deg(ei[1])
    u, dis, invdeg = _k_prep(xp, degp)
    p = _k_hop(u, src, dst)
    w1 = _k_mid(p, u, invdeg)
    q = _k_hop(w1, src, dst)
    return _k_fin(q, w1, dis, W, b)[:N_NODES]


# speedup vs baseline: 2.6940x; 2.6940x over previous
import jax
import jax.numpy as jnp
from jax import lax
from jax.experimental import pallas as pl
from jax.experimental.pallas import tpu as pltpu
from jax.experimental.pallas import tpu_sc as plsc

N_NODES = 10000
N_EDGES = 320000
D = 128
NC = 2
NS = 16
NW = NC * NS
NPAD = 10240
EPW = N_EDGES // NW
EPWP = 10240
CHK = 200
RZ = 200
RPS = NPAD // NS


def _mesh():
    return plsc.VectorSubcoreMesh(core_axis_name="c", subcore_axis_name="s")


def _deg_body(dst_hbm, out_hbm, dbuf, ones, zv, shared):
    c = lax.axis_index("c")
    s = lax.axis_index("s")
    w = c * NS + s

    def fill(i, _):
        ones[pl.ds(i * 16, 16)] = jnp.full((16,), 1.0, jnp.float32)
        zv[pl.ds(i * 16, 16)] = jnp.zeros((16,), jnp.float32)
        return 0

    lax.fori_loop(0, NPAD // 16, fill, 0)

    @pl.when(s == 0)
    def _():
        pltpu.sync_copy(zv, shared)

    plsc.subcore_barrier()
    pltpu.sync_copy(dst_hbm.at[pl.ds(w * EPW, EPW)], dbuf)
    pltpu.sync_copy(ones.at[pl.ds(0, EPW)], shared.at[dbuf], add=True)
    plsc.subcore_barrier()

    @pl.when(s == 0)
    def _():
        pltpu.sync_copy(shared, out_hbm.at[c])


def _k_deg(dst):
    f = pl.kernel(
        _deg_body,
        out_type=jax.ShapeDtypeStruct((NC, NPAD), jnp.float32),
        mesh=_mesh(),
        scratch_types=[
            pltpu.VMEM((EPW,), jnp.int32),
            pltpu.VMEM((NPAD,), jnp.float32),
            pltpu.VMEM((NPAD,), jnp.float32),
            pltpu.VMEM_SHARED((NPAD,), jnp.float32),
        ],
    )
    return f(dst)


HCHK = 128
NCH = EPWP // HCHK
LOOPN = NCH // 2
TAIL = NCH % 2


def _hop_body(u_hbm, src_hbm, dst_hbm, out_hbm, rows0, rows1, sb0, sb1,
              db0, db1, shared, gsem0, gsem1, ssem0, ssem1, isem0, isem1):
    c = lax.axis_index("c")
    s = lax.axis_index("s")
    w = c * NS + s
    ebase = w * EPWP
    zf = jnp.zeros((16,), jnp.float32)
    rbufs = (rows0, rows1)
    sbufs = (sb0, sb1)
    dbufs = (db0, db1)
    gsems = (gsem0, gsem1)
    ssems = (ssem0, ssem1)
    isems = (isem0, isem1)

    def zr(r, _):
        for f in range(D // 16):
            rows0[r, pl.ds(f * 16, 16)] = zf
        return 0

    lax.fori_loop(0, HCHK, zr, 0)

    def zs(i, _):
        pltpu.sync_copy(rows0, shared.at[pl.ds(s * RPS + i * HCHK, HCHK)])
        return 0

    lax.fori_loop(0, RPS // HCHK, zs, 0)
    plsc.subcore_barrier()

    def _idx_start(k, b):
        pltpu.async_copy(src_hbm.at[pl.ds(ebase + k * HCHK, HCHK)],
                         sbufs[b], isems[b])
        pltpu.async_copy(dst_hbm.at[pl.ds(ebase + k * HCHK, HCHK)],
                         dbufs[b], isems[b])

    def _gather_start(b):
        return pltpu.async_copy(u_hbm.at[sbufs[b]], rbufs[b], gsems[b])

    def _gather_wait(b):
        pltpu.make_async_copy(u_hbm.at[sbufs[b]], rbufs[b], gsems[b]).wait()

    def _scatter_start(b):
        return pltpu.async_copy(rbufs[b], shared.at[dbufs[b]], ssems[b],
                                add=True)

    def _scatter_wait(b):
        pltpu.make_async_copy(rbufs[b], shared.at[dbufs[b]], ssems[b]).wait()

    def _idx_wait(b):
        pltpu.make_async_copy(src_hbm.at[pl.ds(0, HCHK)], sbufs[b],
                              isems[b]).wait()
        pltpu.make_async_copy(dst_hbm.at[pl.ds(0, HCHK)], dbufs[b],
                              isems[b]).wait()

    _idx_start(0, 0)
    _idx_wait(0)
    _gather_start(0)
    _idx_start(1, 1)
    _idx_wait(1)
    _gather_start(1)

    def body(t, _):
        _gather_wait(0)
        _scatter_start(0)
        _gather_wait(1)
        _scatter_start(1)

        @pl.when(t < LOOPN - 1)
        def _():
            _scatter_wait(0)
            _idx_start(2 * t + 2, 0)
            _idx_wait(0)
            _gather_start(0)
            _scatter_wait(1)
            _idx_start(2 * t + 3, 1)
            _idx_wait(1)
            _gather_start(1)

        @pl.when(t >= LOOPN - 1)
        def _():
            _scatter_wait(0)
            _scatter_wait(1)

        return 0

    lax.fori_loop(0, LOOPN, body, 0)
    if TAIL:
        _idx_start(NCH - 1, 0)
        _idx_wait(0)
        _gather_start(0)
        _gather_wait(0)
        _scatter_start(0)
        _scatter_wait(0)
    plsc.subcore_barrier()
    pltpu.sync_copy(shared.at[pl.ds(s * RPS, RPS)],
                    out_hbm.at[c, pl.ds(s * RPS, RPS)])


def _k_hop(u, src, dst):
    f = pl.kernel(
        _hop_body,
        out_type=jax.ShapeDtypeStruct((NC, NPAD, D), jnp.float32),
        mesh=_mesh(),
        scratch_types=[
            pltpu.VMEM((HCHK, D), jnp.float32),
            pltpu.VMEM((HCHK, D), jnp.float32),
            pltpu.VMEM((HCHK,), jnp.int32),
            pltpu.VMEM((HCHK,), jnp.int32),
            pltpu.VMEM((HCHK,), jnp.int32),
            pltpu.VMEM((HCHK,), jnp.int32),
            pltpu.VMEM_SHARED((NPAD, D), jnp.float32),
            pltpu.SemaphoreType.DMA,
            pltpu.SemaphoreType.DMA,
            pltpu.SemaphoreType.DMA,
            pltpu.SemaphoreType.DMA,
            pltpu.SemaphoreType.DMA,
            pltpu.SemaphoreType.DMA,
        ],
    )
    return f(u, src, dst)


def _prep_kernel(x_ref, dp_ref, u_ref, dis_ref, inv_ref):
    deg = dp_ref[0] + dp_ref[1] + 1.0
    dis = lax.rsqrt(deg)
    dis_ref[...] = dis
    inv_ref[...] = 1.0 / deg
    u_ref[...] = x_ref[...] * dis[:, None]


def _k_prep(xp, degp):
    return pl.pallas_call(
        _prep_kernel,
        out_shape=[
            jax.ShapeDtypeStruct((NPAD, D), jnp.float32),
            jax.ShapeDtypeStruct((NPAD,), jnp.float32),
            jax.ShapeDtypeStruct((NPAD,), jnp.float32),
        ],
    )(xp, degp)


def _mid_kernel(p_ref, u_ref, inv_ref, w_ref):
    w_ref[...] = (p_ref[0] + p_ref[1] + u_ref[...]) * inv_ref[...][:, None]


def _k_mid(p, u, invdeg):
    return pl.pallas_call(
        _mid_kernel,
        out_shape=jax.ShapeDtypeStruct((NPAD, D), jnp.float32),
    )(p, u, invdeg)


def _fin_kernel(q_ref, w_ref, dis_ref, wt_ref, b_ref, o_ref):
    h2 = (q_ref[0] + q_ref[1] + w_ref[...]) * dis_ref[...][:, None]
    o_ref[...] = (
        jnp.dot(h2, wt_ref[...], preferred_element_type=jnp.float32)
        + b_ref[...]
    )


def _k_fin(q, w1, dis, W, b):
    return pl.pallas_call(
        _fin_kernel,
        out_shape=jax.ShapeDtypeStruct((NPAD, D), jnp.float32),
    )(q, w1, dis, W.T, b.reshape(1, D))


def kernel(x, edge_index, W, b):
    ei = edge_index.astype(jnp.int32)
    xp = jnp.pad(x, ((0, NPAD - N_NODES), (0, 0)))
    padv = jnp.broadcast_to(
        N_NODES + jnp.arange(EPWP - EPW, dtype=jnp.int32), (NW, EPWP - EPW))
    src = jnp.concatenate([ei[0].reshape(NW, EPW), padv], axis=1).reshape(-1)
    dst = jnp.concatenate([ei[1].reshape(NW, EPW), padv], axis=1).reshape(-1)
    degp = _k_deg(ei[1])
    u, dis, invdeg = _k_prep(xp, degp)
    p = _k_hop(u, src, dst)
    w1 = _k_mid(p, u, invdeg)
    q = _k_hop(w1, src, dst)
    return _k_fin(q, w1, dis, W, b)[:N_NODES]


# speedup vs baseline: 2.7020x; 1.0030x over previous
import jax
import jax.numpy as jnp
from jax import lax
from jax.experimental import pallas as pl
from jax.experimental.pallas import tpu as pltpu
from jax.experimental.pallas import tpu_sc as plsc

N_NODES = 10000
N_EDGES = 320000
D = 128
NC = 2
NS = 16
NW = NC * NS
NPAD = 10240
EPW = N_EDGES // NW
EPWP = 10240
CHK = 200
RZ = 200
RPS = NPAD // NS


def _mesh():
    return plsc.VectorSubcoreMesh(core_axis_name="c", subcore_axis_name="s")


def _deg_body(dst_hbm, out_hbm, dbuf, ones, zv, shared):
    c = lax.axis_index("c")
    s = lax.axis_index("s")
    w = c * NS + s

    def fill(i, _):
        ones[pl.ds(i * 16, 16)] = jnp.full((16,), 1.0, jnp.float32)
        zv[pl.ds(i * 16, 16)] = jnp.zeros((16,), jnp.float32)
        return 0

    lax.fori_loop(0, NPAD // 16, fill, 0)

    @pl.when(s == 0)
    def _():
        pltpu.sync_copy(zv, shared)

    plsc.subcore_barrier()
    pltpu.sync_copy(dst_hbm.at[pl.ds(w * EPW, EPW)], dbuf)
    pltpu.sync_copy(ones.at[pl.ds(0, EPW)], shared.at[dbuf], add=True)
    plsc.subcore_barrier()

    @pl.when(s == 0)
    def _():
        pltpu.sync_copy(shared, out_hbm.at[c])


def _k_deg(dst):
    f = pl.kernel(
        _deg_body,
        out_type=jax.ShapeDtypeStruct((NC, NPAD), jnp.float32),
        mesh=_mesh(),
        scratch_types=[
            pltpu.VMEM((EPW,), jnp.int32),
            pltpu.VMEM((NPAD,), jnp.float32),
            pltpu.VMEM((NPAD,), jnp.float32),
            pltpu.VMEM_SHARED((NPAD,), jnp.float32),
        ],
    )
    return f(dst)


HCHK = 128
NCH = EPWP // HCHK
LOOPN = NCH // 2
TAIL = NCH % 2


def _hop_body(u_hbm, src_hbm, dst_hbm, out_hbm, rows0, rows1, sb0, sb1,
              db0, db1, shared, gsem0, gsem1, ssem0, ssem1, isem0, isem1):
    c = lax.axis_index("c")
    s = lax.axis_index("s")
    w = c * NS + s
    ebase = w * EPWP
    zf = jnp.zeros((16,), jnp.float32)
    rbufs = (rows0, rows1)
    sbufs = (sb0, sb1)
    dbufs = (db0, db1)
    gsems = (gsem0, gsem1)
    ssems = (ssem0, ssem1)
    isems = (isem0, isem1)

    def zr(r, _):
        for f in range(D // 16):
            rows0[r, pl.ds(f * 16, 16)] = zf
        return 0

    lax.fori_loop(0, HCHK, zr, 0)

    def zs(i, _):
        pltpu.sync_copy(rows0, shared.at[pl.ds(s * RPS + i * HCHK, HCHK)])
        return 0

    lax.fori_loop(0, RPS // HCHK, zs, 0)
    plsc.subcore_barrier()

    def _idx_start(k, b):
        pltpu.async_copy(src_hbm.at[pl.ds(ebase + k * HCHK, HCHK)],
                         sbufs[b], isems[b])
        pltpu.async_copy(dst_hbm.at[pl.ds(ebase + k * HCHK, HCHK)],
                         dbufs[b], isems[b])

    def _gather_start(b):
        return pltpu.async_copy(u_hbm.at[sbufs[b]], rbufs[b], gsems[b])

    def _gather_wait(b):
        pltpu.make_async_copy(u_hbm.at[sbufs[b]], rbufs[b], gsems[b]).wait()

    def _scatter_start(b):
        return pltpu.async_copy(rbufs[b], shared.at[dbufs[b]], ssems[b],
                                add=True)

    def _scatter_wait(b):
        pltpu.make_async_copy(rbufs[b], shared.at[dbufs[b]], ssems[b]).wait()

    def _idx_wait(b):
        pltpu.make_async_copy(src_hbm.at[pl.ds(0, HCHK)], sbufs[b],
                              isems[b]).wait()
        pltpu.make_async_copy(dst_hbm.at[pl.ds(0, HCHK)], dbufs[b],
                              isems[b]).wait()

    _idx_start(0, 0)
    _idx_wait(0)
    _gather_start(0)
    _idx_start(1, 1)
    _idx_wait(1)
    _gather_start(1)

    def body(t, _):
        _gather_wait(0)
        _scatter_start(0)
        _gather_wait(1)
        _scatter_start(1)

        @pl.when(t < LOOPN - 1)
        def _():
            _scatter_wait(0)
            _idx_start(2 * t + 2, 0)
            _idx_wait(0)
            _gather_start(0)
            _scatter_wait(1)
            _idx_start(2 * t + 3, 1)
            _idx_wait(1)
            _gather_start(1)

        @pl.when(t >= LOOPN - 1)
        def _():
            _scatter_wait(0)
            _scatter_wait(1)

        return 0

    lax.fori_loop(0, LOOPN, body, 0)
    if TAIL:
        _idx_start(NCH - 1, 0)
        _idx_wait(0)
        _gather_start(0)
        _gather_wait(0)
        _scatter_start(0)
        _scatter_wait(0)
    plsc.subcore_barrier()
    pltpu.sync_copy(shared.at[pl.ds(s * RPS, RPS)],
                    out_hbm.at[c, pl.ds(s * RPS, RPS)])


def _k_hop(u, src, dst):
    f = pl.kernel(
        _hop_body,
        out_type=jax.ShapeDtypeStruct((NC, NPAD, D), jnp.float32),
        mesh=_mesh(),
        scratch_types=[
            pltpu.VMEM((HCHK, D), jnp.float32),
            pltpu.VMEM((HCHK, D), jnp.float32),
            pltpu.VMEM((HCHK,), jnp.int32),
            pltpu.VMEM((HCHK,), jnp.int32),
            pltpu.VMEM((HCHK,), jnp.int32),
            pltpu.VMEM((HCHK,), jnp.int32),
            pltpu.VMEM_SHARED((NPAD, D), jnp.float32),
            pltpu.SemaphoreType.DMA,
            pltpu.SemaphoreType.DMA,
            pltpu.SemaphoreType.DMA,
            pltpu.SemaphoreType.DMA,
            pltpu.SemaphoreType.DMA,
            pltpu.SemaphoreType.DMA,
        ],
    )
    return f(u, src, dst)


def _prep_kernel(x_ref, dp_ref, u_ref, dis_ref, inv_ref):
    deg = dp_ref[0] + dp_ref[1] + 1.0
    dis = lax.rsqrt(deg)
    dis_ref[...] = dis
    inv_ref[...] = 1.0 / deg
    u_ref[...] = x_ref[...] * dis[:, None]


def _k_prep(xp, degp):
    return pl.pallas_call(
        _prep_kernel,
        out_shape=[
            jax.ShapeDtypeStruct((NPAD, D), jnp.float32),
            jax.ShapeDtypeStruct((NPAD,), jnp.float32),
            jax.ShapeDtypeStruct((NPAD,), jnp.float32),
        ],
    )(xp, degp)


def _mid_kernel(p_ref, u_ref, inv_ref, w_ref):
    w_ref[...] = (p_ref[0] + p_ref[1] + u_ref[...]) * inv_ref[...]


def _k_mid(p, u, invdeg):
    blk = 2048
    return pl.pallas_call(
        _mid_kernel,
        grid=(NPAD // blk,),
        in_specs=[
            pl.BlockSpec((NC, blk, D), lambda i: (0, i, 0)),
            pl.BlockSpec((blk, D), lambda i: (i, 0)),
            pl.BlockSpec((blk, 1), lambda i: (i, 0)),
        ],
        out_specs=pl.BlockSpec((blk, D), lambda i: (i, 0)),
        out_shape=jax.ShapeDtypeStruct((NPAD, D), jnp.float32),
    )(p, u, invdeg.reshape(NPAD, 1))


def _fin_kernel(q_ref, w_ref, dis_ref, wt_ref, b_ref, o_ref):
    h2 = (q_ref[0] + q_ref[1] + w_ref[...]) * dis_ref[...]
    o_ref[...] = (
        jnp.dot(h2, wt_ref[...], preferred_element_type=jnp.float32)
        + b_ref[...]
    )


def _k_fin(q, w1, dis, W, b):
    blk = 2000
    return pl.pallas_call(
        _fin_kernel,
        grid=(N_NODES // blk,),
        in_specs=[
            pl.BlockSpec((NC, blk, D), lambda i: (0, i, 0)),
            pl.BlockSpec((blk, D), lambda i: (i, 0)),
            pl.BlockSpec((blk, 1), lambda i: (i, 0)),
            pl.BlockSpec((D, D), lambda i: (0, 0)),
            pl.BlockSpec((1, D), lambda i: (0, 0)),
        ],
        out_specs=pl.BlockSpec((blk, D), lambda i: (i, 0)),
        out_shape=jax.ShapeDtypeStruct((N_NODES, D), jnp.float32),
    )(q, w1, dis.reshape(NPAD, 1), W.T, b.reshape(1, D))


def kernel(x, edge_index, W, b):
    ei = edge_index.astype(jnp.int32)
    xp = jnp.pad(x, ((0, NPAD - N_NODES), (0, 0)))
    padv = jnp.broadcast_to(
        N_NODES + jnp.arange(EPWP - EPW, dtype=jnp.int32), (NW, EPWP - EPW))
    src = jnp.concatenate([ei[0].reshape(NW, EPW), padv], axis=1).reshape(-1)
    dst = jnp.concatenate([ei[1].reshape(NW, EPW), padv], axis=1).reshape(-1)
    degp = _k_deg(ei[1])
    u, dis, invdeg = _k_prep(xp, degp)
    p = _k_hop(u, src, dst)
    w1 = _k_mid(p, u, invdeg)
    q = _k_hop(w1, src, dst)
    return _k_fin(q, w1, dis, W, b)
